# baseline (speedup 1.0000x reference)
import jax
import jax.numpy as jnp
from jax.experimental import pallas as pl
from jax.experimental.pallas import tpu as pltpu

TOKENS = 16384
HIDDEN = 4096
CTX = HIDDEN // 4
EXPERTS = 64
BM = 512
CH = CTX // 2


def _gelu(h):
    return 0.5 * h * (1.0 + jax.lax.erf(h * 0.7071067811865476))


def _router_body(t_ref, x_ref, w1_ref, b1_ref, w2_ref, out_ref,
                 w1b_ref, w2b_ref):
    @pl.when(pl.program_id(0) == 0)
    def _cast_weights():
        w1b_ref[...] = w1_ref[...].astype(jnp.bfloat16)
        w2b_ref[...] = w2_ref[...].astype(jnp.bfloat16)

    inv_t = 1.0 / t_ref[0]
    xb = x_ref[...].astype(jnp.bfloat16)
    hA = jnp.dot(xb, w1b_ref[:, :CH], preferred_element_type=jnp.float32)
    hB = jnp.dot(xb, w1b_ref[:, CH:], preferred_element_type=jnp.float32)
    gA = _gelu(hA + b1_ref[:, :CH])
    pA = jnp.dot(gA.astype(jnp.bfloat16), w2b_ref[:CH, :],
                 preferred_element_type=jnp.float32)
    gB = _gelu(hB + b1_ref[:, CH:])
    pB = jnp.dot(gB.astype(jnp.bfloat16), w2b_ref[CH:, :],
                 preferred_element_type=jnp.float32)
    out_ref[...] = (pA + pB) * inv_t


def kernel(x, W1, b1, W2, temperature):
    b1r = b1.reshape(1, CTX)
    grid = (TOKENS // BM,)
    return pl.pallas_call(
        _router_body,
        grid=grid,
        in_specs=[
            pl.BlockSpec(memory_space=pltpu.SMEM),
            pl.BlockSpec((BM, HIDDEN), lambda i: (i, 0)),
            pl.BlockSpec((HIDDEN, CTX), lambda i: (0, 0)),
            pl.BlockSpec((1, CTX), lambda i: (0, 0)),
            pl.BlockSpec((CTX, EXPERTS), lambda i: (0, 0)),
        ],
        out_specs=pl.BlockSpec((BM, EXPERTS), lambda i: (i, 0)),
        out_shape=jax.ShapeDtypeStruct((TOKENS, EXPERTS), jnp.float32),
        scratch_shapes=[
            pltpu.VMEM((HIDDEN, CTX), jnp.bfloat16),
            pltpu.VMEM((CTX, EXPERTS), jnp.bfloat16),
        ],
        compiler_params=pltpu.CompilerParams(
            dimension_semantics=("arbitrary",),
            vmem_limit_bytes=64 * 1024 * 1024,
        ),
    )(temperature, x, W1, b1r, W2)


# speedup vs baseline: 1.0551x; 1.0551x over previous
import jax
import jax.numpy as jnp
from jax.experimental import pallas as pl
from jax.experimental.pallas import tpu as pltpu

TOKENS = 16384
HIDDEN = 4096
CTX = HIDDEN // 4
EXPERTS = 64
BM = 512


def _router_body(t_ref, x_ref, w1_ref, b1_ref, w2_ref, out_ref,
                 w1b_ref, w2b_ref):
    @pl.when(pl.program_id(0) == 0)
    def _cast_weights():
        w1b_ref[...] = w1_ref[...].astype(jnp.bfloat16)
        w2b_ref[...] = (w2_ref[...] * (0.5 / t_ref[0])).astype(jnp.bfloat16)

    xb = x_ref[...].astype(jnp.bfloat16)
    h = jnp.dot(xb, w1b_ref[...], preferred_element_type=jnp.float32)
    h = h + b1_ref[...]
    g2 = h * (1.0 + jax.lax.erf(h * 0.7071067811865476))
    out_ref[...] = jnp.dot(g2.astype(jnp.bfloat16), w2b_ref[...],
                           preferred_element_type=jnp.float32)


def kernel(x, W1, b1, W2, temperature):
    b1r = b1.reshape(1, CTX)
    grid = (TOKENS // BM,)
    return pl.pallas_call(
        _router_body,
        grid=grid,
        in_specs=[
            pl.BlockSpec(memory_space=pltpu.SMEM),
            pl.BlockSpec((BM, HIDDEN), lambda i: (i, 0)),
            pl.BlockSpec((HIDDEN, CTX), lambda i: (0, 0)),
            pl.BlockSpec((1, CTX), lambda i: (0, 0)),
            pl.BlockSpec((CTX, EXPERTS), lambda i: (0, 0)),
        ],
        out_specs=pl.BlockSpec((BM, EXPERTS), lambda i: (i, 0)),
        out_shape=jax.ShapeDtypeStruct((TOKENS, EXPERTS), jnp.float32),
        scratch_shapes=[
            pltpu.VMEM((HIDDEN, CTX), jnp.bfloat16),
            pltpu.VMEM((CTX, EXPERTS), jnp.bfloat16),
        ],
        compiler_params=pltpu.CompilerParams(
            dimension_semantics=("arbitrary",),
            vmem_limit_bytes=64 * 1024 * 1024,
        ),
    )(temperature, x, W1, b1r, W2)
